# Initial kernel scaffold; baseline (speedup 1.0000x reference)
import jax
import jax.numpy as jnp
from jax.experimental import pallas as pl


def kernel(input_ids, position_ids, word_table, pos_table):
    raise NotImplementedError("write your pallas kernel here")


# speedup vs baseline: 1.2903x; 1.2903x over previous
import functools

import jax
import jax.numpy as jnp
from jax import lax
from jax.experimental import pallas as pl
from jax.experimental.pallas import tpu as pltpu
from jax.experimental.pallas import tpu_sc as plsc

_B, _S, _H = 4, 8192, 1024
_N = _B * _S
_NC, _NS = 2, 16
_NW = _NC * _NS
_TOKW = _N // _NW
_CHUNK = 32
_NCH = _TOKW // _CHUNK
_LANES = 16

_mesh = plsc.VectorSubcoreMesh(core_axis_name="c", subcore_axis_name="s")


@functools.partial(
    pl.kernel,
    out_type=jax.ShapeDtypeStruct((_N, _H), jnp.float32),
    mesh=_mesh,
    scratch_types=[
        pltpu.VMEM((_NCH, _CHUNK), jnp.int32),
        pltpu.VMEM((_NCH, _CHUNK), jnp.int32),
        pltpu.VMEM((_CHUNK, _H), jnp.float32),
        pltpu.VMEM((_CHUNK, _H), jnp.float32),
        pltpu.SemaphoreType.DMA,
    ],
)
def _embed(ids_hbm, pos_hbm, wt_hbm, pt_hbm, out_hbm, widx, pidx, bufw, bufp, sem):
    wid = lax.axis_index("s") * _NC + lax.axis_index("c")
    pltpu.sync_copy(ids_hbm.at[wid], widx)
    pltpu.sync_copy(pos_hbm.at[wid], pidx)

    @pl.loop(0, _NCH)
    def _chunk(c):
        cw = pltpu.async_copy(wt_hbm.at[widx.at[c]], bufw, sem)
        cp = pltpu.async_copy(pt_hbm.at[pidx.at[c]], bufp, sem)
        cw.wait()
        cp.wait()

        @pl.loop(0, _CHUNK)
        def _row(r):
            for j in range(_H // _LANES):
                sl = pl.ds(j * _LANES, _LANES)
                bufw[r, sl] += bufp[r, sl]

        row0 = wid * _TOKW + c * _CHUNK
        pltpu.sync_copy(bufw, out_hbm.at[pl.ds(row0, _CHUNK)])


@jax.jit
def kernel(input_ids, position_ids, word_table, pos_table):
    ids = input_ids.astype(jnp.int32).reshape(_NW, _NCH, _CHUNK)
    pos = position_ids.astype(jnp.int32).reshape(_NW, _NCH, _CHUNK)
    out = _embed(ids, pos, word_table, pos_table)
    return out.reshape(_B, _S, _H)


# speedup vs baseline: 2.0268x; 1.5708x over previous
import functools

import jax
import jax.numpy as jnp
from jax import lax
from jax.experimental import pallas as pl
from jax.experimental.pallas import tpu as pltpu
from jax.experimental.pallas import tpu_sc as plsc

_B, _S, _H = 4, 8192, 1024
_N = _B * _S
_NC, _NS = 2, 16
_NW = _NC * _NS
_TOKW = _N // _NW
_CHUNK = 16
_NCH = _TOKW // _CHUNK
_LANES = 16

_mesh = plsc.VectorSubcoreMesh(core_axis_name="c", subcore_axis_name="s")


@functools.partial(
    pl.kernel,
    out_type=jax.ShapeDtypeStruct((_N, _H), jnp.float32),
    mesh=_mesh,
    scratch_types=[
        pltpu.VMEM((_NCH, _CHUNK), jnp.int32),
        pltpu.VMEM((_NCH, _CHUNK), jnp.int32),
        pltpu.VMEM((_CHUNK, _H), jnp.float32),
        pltpu.VMEM((_CHUNK, _H), jnp.float32),
        pltpu.VMEM((_CHUNK, _H), jnp.float32),
        pltpu.VMEM((_CHUNK, _H), jnp.float32),
        pltpu.SemaphoreType.DMA,
        pltpu.SemaphoreType.DMA,
        pltpu.SemaphoreType.DMA,
        pltpu.SemaphoreType.DMA,
    ],
)
def _embed(ids_hbm, pos_hbm, wt_hbm, pt_hbm, out_hbm,
           widx, pidx, bufw0, bufp0, bufw1, bufp1,
           semg0, semg1, semst0, semst1):
    wid = lax.axis_index("s") * _NC + lax.axis_index("c")
    pltpu.sync_copy(ids_hbm.at[wid], widx)
    pltpu.sync_copy(pos_hbm.at[wid], pidx)

    bufs = ((bufw0, bufp0, semg0, semst0), (bufw1, bufp1, semg1, semst1))

    pltpu.async_copy(wt_hbm.at[widx.at[0]], bufw0, semg0)
    pltpu.async_copy(pt_hbm.at[pidx.at[0]], bufp0, semg0)

    @pl.loop(0, _NCH, step=2)
    def _pair(c0):
        for k in range(2):
            c = c0 + k
            bufw, bufp, semg, semst = bufs[k]
            o_bufw, o_bufp, o_semg, o_semst = bufs[1 - k]

            pltpu.make_async_copy(wt_hbm.at[pl.ds(0, _CHUNK)], bufw, semg).wait()
            pltpu.make_async_copy(wt_hbm.at[pl.ds(0, _CHUNK)], bufp, semg).wait()

            @pl.when(c >= 1)
            def _drain_store():
                pltpu.make_async_copy(
                    o_bufw, out_hbm.at[pl.ds(0, _CHUNK)], o_semst).wait()

            @pl.when(c + 1 < _NCH)
            def _fire_next():
                pltpu.async_copy(wt_hbm.at[widx.at[c + 1]], o_bufw, o_semg)
                pltpu.async_copy(pt_hbm.at[pidx.at[c + 1]], o_bufp, o_semg)

            @pl.loop(0, _CHUNK)
            def _row(r):
                for j in range(_H // _LANES):
                    sl = pl.ds(j * _LANES, _LANES)
                    bufw[r, sl] += bufp[r, sl]

            row0 = wid * _TOKW + c * _CHUNK
            pltpu.async_copy(bufw, out_hbm.at[pl.ds(row0, _CHUNK)], semst)

    pltpu.make_async_copy(bufw1, out_hbm.at[pl.ds(0, _CHUNK)], semst1).wait()


@jax.jit
def kernel(input_ids, position_ids, word_table, pos_table):
    ids = input_ids.astype(jnp.int32).reshape(_NW, _NCH, _CHUNK)
    pos = position_ids.astype(jnp.int32).reshape(_NW, _NCH, _CHUNK)
    out = _embed(ids, pos, word_table, pos_table)
    return out.reshape(_B, _S, _H)
